# baseline (speedup 1.0000x reference)
import functools

import jax
import jax.numpy as jnp
from jax import lax
from jax.experimental import pallas as pl
from jax.experimental.pallas import tpu as pltpu
from jax.experimental.pallas import tpu_sc as plsc

B0, B1 = 16384, 50
D = 64
V = 1000000
NW = 32
UNITS = B1 * (B0 // 128)
UNITS_PER_W = UNITS // NW

_mesh = plsc.VectorSubcoreMesh(core_axis_name="c", subcore_axis_name="s")


@functools.partial(
    pl.kernel,
    mesh=_mesh,
    compiler_params=pltpu.CompilerParams(needs_layout_passes=False),
    out_type=jax.ShapeDtypeStruct((B1, D, B0), jnp.float32),
    scratch_types=[
        pltpu.VMEM((2, 128), jnp.int32),
        pltpu.VMEM((2, 128), jnp.int32),
        pltpu.VMEM((2, 128), jnp.int32),
        pltpu.VMEM((2, 128, 128), jnp.float32),
        pltpu.VMEM((2, D, 128), jnp.float32),
        pltpu.SemaphoreType.DMA,
        pltpu.SemaphoreType.DMA,
        pltpu.SemaphoreType.DMA,
        pltpu.SemaphoreType.DMA,
        pltpu.SemaphoreType.DMA,
        pltpu.SemaphoreType.DMA,
    ],
)
def _gather(xt_hbm, w2_hbm, out_hbm, idx_v, q_v, p_v, rows_v, tile_v,
            is0, is1, gs0, gs1, os0, os1):
    isem = (is0, is1)
    gsem = (gs0, gs1)
    osem = (os0, os1)
    wid = lax.axis_index("s") * 2 + lax.axis_index("c")
    u0 = wid * UNITS_PER_W
    lanes = lax.iota(jnp.int32, 16)

    def idx_src(i):
        u = u0 + i
        s = u // 128
        tb = u - s * 128
        return xt_hbm.at[s, pl.ds(tb * 128, 128)]

    def fire_idx(i, bb):
        pltpu.async_copy(idx_src(i), idx_v.at[bb], isem[bb])

    def wait_idx(i, bb):
        pltpu.make_async_copy(idx_src(i), idx_v.at[bb], isem[bb]).wait()

    def compute_qp(bb):
        for g in range(8):
            iv = idx_v[bb, pl.ds(g * 16, 16)]
            q_v[bb, pl.ds(g * 16, 16)] = lax.shift_right_logical(iv, 1)
            p_v[bb, pl.ds(g * 16, 16)] = lax.mul(lax.bitwise_and(iv, 1), 64)

    def fire_gather(bb):
        pltpu.async_copy(w2_hbm.at[q_v.at[bb]], rows_v.at[bb], gsem[bb])

    def wait_gather(bb):
        pltpu.make_async_copy(w2_hbm.at[q_v.at[bb]], rows_v.at[bb],
                              gsem[bb]).wait()

    def out_dst(i):
        u = u0 + i
        s = u // 128
        tb = u - s * 128
        return out_hbm.at[s, :, pl.ds(tb * 128, 128)]

    def fire_out(i, bb):
        pltpu.async_copy(tile_v.at[bb], out_dst(i), osem[bb])

    def wait_out(i, bb):
        pltpu.make_async_copy(tile_v.at[bb], out_dst(i), osem[bb]).wait()

    full = lanes < 16

    def transpose(bb):
        @plsc.parallel_loop(0, 8, 1, unroll=2)
        def _g(g):
            g16 = g * 16
            col0 = p_v[bb, pl.ds(g16, 16)]
            toks = lanes + g16
            for f in range(D):
                vals = plsc.load_gather(rows_v.at[bb], [toks, col0 + f])
                plsc.store_compressed(tile_v.at[bb, f, pl.ds(g16, 16)],
                                      vals, mask=full)

    fire_idx(0, 0)
    wait_idx(0, 0)
    compute_qp(0)
    fire_gather(0)
    fire_idx(1, 1)

    def body(i, carry):
        for b in range(2):
            ii = i * 2 + b
            nb = 1 - b

            @pl.when(ii < UNITS_PER_W - 1)
            def _prefetch():
                wait_idx(ii + 1, nb)
                compute_qp(nb)
                fire_gather(nb)

            @pl.when(ii < UNITS_PER_W - 2)
            def _stage_next():
                fire_idx(ii + 2, b)

            @pl.when(ii >= 2)
            def _drain_out():
                wait_out(ii - 2, b)

            wait_gather(b)
            transpose(b)
            fire_out(ii, b)
        return carry

    lax.fori_loop(0, UNITS_PER_W // 2, body, 0)

    wait_out(UNITS_PER_W - 2, 0)
    wait_out(UNITS_PER_W - 1, 1)


def kernel(x, weight):
    xt = x.astype(jnp.int32).T
    w2 = weight.reshape(V // 2, 128)
    out_k = _gather(xt, w2)
    return jnp.transpose(out_k, (2, 0, 1))


# speedup vs baseline: 1.2661x; 1.2661x over previous
import functools

import jax
import jax.numpy as jnp
from jax import lax
from jax.experimental import pallas as pl
from jax.experimental.pallas import tpu as pltpu
from jax.experimental.pallas import tpu_sc as plsc

B0, B1 = 16384, 50
D = 64
TOKENS = B0 * B1
NW = 32
IDXW = 128
NROWS = TOKENS // IDXW
ROWS_PER_W = NROWS // NW
CHUNK_IR = 5
NCHUNK = ROWS_PER_W // CHUNK_IR
NBUF = 2
NROUNDS = NCHUNK // NBUF

_mesh = plsc.VectorSubcoreMesh(core_axis_name="c", subcore_axis_name="s")


@functools.partial(
    pl.kernel,
    mesh=_mesh,
    compiler_params=pltpu.CompilerParams(use_tc_tiling_on_sc=False),
    out_type=jax.ShapeDtypeStruct((NROWS, IDXW, D), jnp.float32),
    scratch_types=[
        pltpu.VMEM((ROWS_PER_W, IDXW), jnp.int32),
        pltpu.VMEM((NBUF, CHUNK_IR, IDXW, D), jnp.float32),
        pltpu.SemaphoreType.DMA,
        pltpu.SemaphoreType.DMA,
        pltpu.SemaphoreType.DMA,
        pltpu.SemaphoreType.DMA,
    ],
)
def _gather(idx_hbm, table_hbm, out_hbm, idx_all, rows, g0, g1, o0, o1):
    gsem = (g0, g1)
    osem = (o0, o1)
    wid = lax.axis_index("s") * 2 + lax.axis_index("c")
    row0 = wid * ROWS_PER_W

    pltpu.sync_copy(idx_hbm.at[pl.ds(row0, ROWS_PER_W)], idx_all)

    def fire(c, b):
        for j in range(CHUNK_IR):
            pltpu.async_copy(table_hbm.at[idx_all.at[c * CHUNK_IR + j]],
                             rows.at[b, j], gsem[b])

    def drain_gather(c, b):
        for j in range(CHUNK_IR):
            pltpu.make_async_copy(table_hbm.at[idx_all.at[c * CHUNK_IR + j]],
                                  rows.at[b, j], gsem[b]).wait()

    def start_out(c, b):
        pltpu.async_copy(rows.at[b],
                         out_hbm.at[pl.ds(row0 + c * CHUNK_IR, CHUNK_IR)],
                         osem[b])

    def drain_out(c, b):
        pltpu.make_async_copy(rows.at[b],
                              out_hbm.at[pl.ds(row0 + c * CHUNK_IR, CHUNK_IR)],
                              osem[b]).wait()

    fire(0, 0)
    fire(1, 1)
    drain_gather(0, 0)
    start_out(0, 0)

    def round_body(r, carry):
        for b in range(NBUF):
            c = r * NBUF + b
            drain_out(c - NBUF, b)
            fire(c, b)
            pb = 1 - b
            drain_gather(c - 1, pb)
            start_out(c - 1, pb)
        return carry

    lax.fori_loop(1, NROUNDS, round_body, 0)

    last = NCHUNK - 1
    drain_gather(last, 1)
    start_out(last, 1)
    drain_out(last - 1, 0)
    drain_out(last, 1)


def kernel(x, weight):
    idx = x.astype(jnp.int32).reshape(NROWS, IDXW)
    out = _gather(idx, weight)
    return out.reshape(B0, B1, D)
